# baseline (speedup 1.0000x reference)
import functools

import jax
import jax.numpy as jnp
from jax import lax
from jax.experimental import pallas as pl
from jax.experimental.pallas import tpu as pltpu
from jax.experimental.pallas import tpu_sc as plsc

_NC = 2
_NS = 16
_NW = _NC * _NS
_C = 512
_R = _C // 128

_PARAMS = pltpu.CompilerParams(
    needs_layout_passes=False, use_tc_tiling_on_sc=False)
_MESH = plsc.VectorSubcoreMesh(core_axis_name="c", subcore_axis_name="s")


def _worker_id():
  return lax.axis_index("s") * _NC + lax.axis_index("c")


def _make_pack_kernel(n, d):
  hw = d // 2
  rows_w = n // _NW
  rpc = 625
  nchunk = rows_w // rpc

  @functools.partial(
      pl.kernel, mesh=_MESH, compiler_params=_PARAMS,
      out_type=[jax.ShapeDtypeStruct((n, hw), jnp.int32)] * 2,
      scratch_types=[
          pltpu.VMEM((rpc, d), jnp.float32),
          pltpu.VMEM((rpc, d), jnp.float32),
          pltpu.VMEM((rpc, hw), jnp.int32),
          pltpu.VMEM((rpc, hw), jnp.int32),
          pltpu.SemaphoreType.DMA,
          pltpu.SemaphoreType.DMA,
          pltpu.SemaphoreType.DMA,
          pltpu.SemaphoreType.DMA,
      ],
  )
  def pk(eu_hbm, ev_hbm, pu_hbm, pv_hbm, ib0, ib1, ob0, ob1, s0, s1, t0, t1):
    wid = _worker_id()
    base = wid * rows_w
    ib = [ib0, ib1]
    ob = [ob0, ob1]
    isem = [s0, s1]
    osem = [t0, t1]
    srcs = [eu_hbm, ev_hbm]
    dsts = [pu_hbm, pv_hbm]
    jobs = [(t, c) for t in range(2) for c in range(nchunk)]

    def fetch(j, b):
      t, c = jobs[j]
      pltpu.make_async_copy(
          srcs[t].at[pl.ds(base + c * rpc, rpc)], ib[b], isem[b]).start()

    fetch(0, 0)
    for j, (t, c) in enumerate(jobs):
      b = j % 2
      if j + 1 < len(jobs):
        fetch(j + 1, 1 - b)
      pltpu.make_async_copy(
          srcs[t].at[pl.ds(0, rpc)], ib[b], isem[b]).wait()
      if j >= 2:
        tp = jobs[j - 2][0]
        pltpu.make_async_copy(
            ob[b], dsts[tp].at[pl.ds(0, rpc)], osem[b]).wait()
      ibb, obb = ib[b], ob[b]

      @plsc.parallel_loop(0, rpc // 5, 1, unroll=2)
      def rowblk(r0, ibb=ibb, obb=obb):
        rs = [r0 * 5 + i for i in range(5)]
        xs = [ibb[r, pl.ds(0, hw)] for r in rs]
        ys = [ibb[r, pl.ds(hw, hw)] for r in rs]
        ws = [plsc.bitcast(
            plsc.pack(x, y, format=plsc.PackFormat.INTERLEAVED), jnp.int32)
            for x, y in zip(xs, ys)]
        for r, w in zip(rs, ws):
          obb[r, pl.ds(0, hw)] = w
      pltpu.make_async_copy(
          ob[b], dsts[t].at[pl.ds(base + c * rpc, rpc)], osem[b]).start()

    for b in (0, 1):
      pltpu.make_async_copy(
          ob[b], dsts[0].at[pl.ds(0, rpc)], osem[b]).wait()

  return pk


def _make_main_kernel(n, d, e):
  hw = d // 2
  nch = e // _C
  base_n = nch // _NW
  extra = nch - base_n * _NW
  npair = (base_n + 2) // 2

  @functools.partial(
      pl.kernel, mesh=_MESH, compiler_params=_PARAMS,
      out_type=jax.ShapeDtypeStruct((e,), jnp.float32),
      scratch_types=[
          pltpu.VMEM((_C,), jnp.int32),
          pltpu.VMEM((_C,), jnp.int32),
          pltpu.VMEM((_C,), jnp.int32),
          pltpu.VMEM((_C,), jnp.int32),
          pltpu.VMEM((_C, hw), jnp.int32),
          pltpu.VMEM((_C, hw), jnp.int32),
          pltpu.VMEM((_C, hw), jnp.int32),
          pltpu.VMEM((_C, hw), jnp.int32),
          pltpu.VMEM((_C,), jnp.float32),
          pltpu.VMEM((_C,), jnp.float32),
          pltpu.VMEM((4, 16, 17), jnp.float32),
          pltpu.SemaphoreType.DMA,
          pltpu.SemaphoreType.DMA,
          pltpu.SemaphoreType.DMA,
          pltpu.SemaphoreType.DMA,
          pltpu.SemaphoreType.DMA,
          pltpu.SemaphoreType.DMA,
      ],
  )
  def k(ei_hbm, pu_hbm, pv_hbm, out_hbm,
        iu0, iu1, iv0, iv1, ru0, ru1, rv0, rv1, ov0, ov1, s1,
        is0, is1, rs0, rs1, os0, os1):
    iu = [iu0, iu1]
    iv = [iv0, iv1]
    ru = [ru0, ru1]
    rv = [rv0, rv1]
    ov = [ov0, ov1]
    isem = [is0, is1]
    rsem = [rs0, rs1]
    osem = [os0, os1]

    wid = _worker_id()
    n_i = jnp.where(wid < extra, base_n + 1, base_n)
    lanes = lax.iota(jnp.int32, 16)

    def edge0(ci):
      return (wid + ci * _NW) * _C

    def fetch_idx(ci, b):
      eb = edge0(ci)
      pltpu.make_async_copy(
          ei_hbm.at[0, pl.ds(eb, _C)], iu[b], isem[b]).start()
      pltpu.make_async_copy(
          ei_hbm.at[1, pl.ds(eb, _C)], iv[b], isem[b]).start()

    def wait_idx(b):
      pltpu.make_async_copy(ei_hbm.at[0, pl.ds(0, _C)], iu[b], isem[b]).wait()
      pltpu.make_async_copy(ei_hbm.at[1, pl.ds(0, _C)], iv[b], isem[b]).wait()

    def fetch_rows(b):
      for j in range(_R):
        sl = pl.ds(j * 128, 128)
        pltpu.make_async_copy(
            pu_hbm.at[iu[b].at[sl]], ru[b].at[sl], rsem[b]).start()
        pltpu.make_async_copy(
            pv_hbm.at[iv[b].at[sl]], rv[b].at[sl], rsem[b]).start()

    def wait_rows(b):
      pltpu.make_async_copy(pu_hbm.at[pl.ds(0, _C)], ru[b], rsem[b]).wait()
      pltpu.make_async_copy(pv_hbm.at[pl.ds(0, _C)], rv[b], rsem[b]).wait()

    def compute_store(ci, b):
      rub, rvb, ovb = ru[b], rv[b], ov[b]

      @plsc.parallel_loop(0, _C // 16, 1, unroll=2)
      def group(g):
        e0 = g * 16
        t = g & 3
        s1t = s1.at[t]
        for blk in range(4):
          es = [e0 + blk * 4 + i for i in range(4)]
          us = [rub[e, pl.ds(0, hw)] for e in es]
          vs = [rvb[e, pl.ds(0, hw)] for e in es]
          ps = [plsc.bitcast(u, jnp.bfloat16) * plsc.bitcast(v, jnp.bfloat16)
                for u, v in zip(us, vs)]
          hs = [plsc.unpack(p, format=plsc.PackFormat.INTERLEAVED)
                for p in ps]
          for i in range(4):
            s1t[blk * 4 + i, pl.ds(0, 16)] = hs[i][0] + hs[i][1]
        gs = [plsc.load_gather(s1t, [lanes, jnp.full((16,), dd, jnp.int32)])
              for dd in range(16)]
        while len(gs) > 1:
          gs = [gs[i] + gs[i + 1] for i in range(0, len(gs), 2)]
        ovb[pl.ds(e0, 16)] = 1.0 / (1.0 + jnp.exp(-gs[0]))
      pltpu.make_async_copy(
          ov[b], out_hbm.at[pl.ds(edge0(ci), _C)], osem[b]).start()

    def wait_out(b):
      pltpu.make_async_copy(
          ov[b], out_hbm.at[pl.ds(0, _C)], osem[b]).wait()

    fetch_idx(0, 0)
    wait_idx(0)
    fetch_rows(0)

    @pl.when(1 < n_i)
    def _():
      fetch_idx(1, 1)

    def pair(p, carry):
      for b in (0, 1):
        ci = p * 2 + b
        nb = 1 - b

        @pl.when(ci < n_i)
        def _():
          @pl.when(ci + 1 < n_i)
          def _():
            wait_idx(nb)
            fetch_rows(nb)

          wait_rows(b)

          @pl.when(ci + 2 < n_i)
          def _():
            fetch_idx(ci + 2, b)

          @pl.when(ci >= 2)
          def _():
            wait_out(b)

          compute_store(ci, b)
      return carry

    lax.fori_loop(0, npair, pair, 0)
    wait_out(0)
    wait_out(1)

  return k


def kernel(Eu, Ev, edge_index):
  n, d = Eu.shape
  e = edge_index.shape[1]
  pu, pv = _make_pack_kernel(n, d)(Eu, Ev)
  return _make_main_kernel(n, d, e)(edge_index.astype(jnp.int32), pu, pv)


# speedup vs baseline: 1.0073x; 1.0073x over previous
import functools

import jax
import jax.numpy as jnp
from jax import lax
from jax.experimental import pallas as pl
from jax.experimental.pallas import tpu as pltpu
from jax.experimental.pallas import tpu_sc as plsc

_NC = 2
_NS = 16
_NW = _NC * _NS
_C = 512
_R = _C // 128

_PARAMS = pltpu.CompilerParams(
    needs_layout_passes=False, use_tc_tiling_on_sc=False)
_MESH = plsc.VectorSubcoreMesh(core_axis_name="c", subcore_axis_name="s")


def _worker_id():
  return lax.axis_index("s") * _NC + lax.axis_index("c")


def _make_pack_kernel(n, d):
  hw = d // 2
  rows_w = n // _NW
  rpc = 625
  nchunk = rows_w // rpc

  @functools.partial(
      pl.kernel, mesh=_MESH, compiler_params=_PARAMS,
      out_type=[jax.ShapeDtypeStruct((n, hw), jnp.int32)] * 2,
      scratch_types=[
          pltpu.VMEM((rpc, d), jnp.float32),
          pltpu.VMEM((rpc, d), jnp.float32),
          pltpu.VMEM((rpc, hw), jnp.int32),
          pltpu.VMEM((rpc, hw), jnp.int32),
          pltpu.SemaphoreType.DMA,
          pltpu.SemaphoreType.DMA,
          pltpu.SemaphoreType.DMA,
          pltpu.SemaphoreType.DMA,
      ],
  )
  def pk(eu_hbm, ev_hbm, pu_hbm, pv_hbm, ib0, ib1, ob0, ob1, s0, s1, t0, t1):
    wid = _worker_id()
    base = wid * rows_w
    ib = [ib0, ib1]
    ob = [ob0, ob1]
    isem = [s0, s1]
    osem = [t0, t1]
    srcs = [eu_hbm, ev_hbm]
    dsts = [pu_hbm, pv_hbm]
    jobs = [(t, c) for t in range(2) for c in range(nchunk)]

    def fetch(j, b):
      t, c = jobs[j]
      pltpu.make_async_copy(
          srcs[t].at[pl.ds(base + c * rpc, rpc)], ib[b], isem[b]).start()

    fetch(0, 0)
    for j, (t, c) in enumerate(jobs):
      b = j % 2
      if j + 1 < len(jobs):
        fetch(j + 1, 1 - b)
      pltpu.make_async_copy(
          srcs[t].at[pl.ds(0, rpc)], ib[b], isem[b]).wait()
      if j >= 2:
        tp = jobs[j - 2][0]
        pltpu.make_async_copy(
            ob[b], dsts[tp].at[pl.ds(0, rpc)], osem[b]).wait()
      ibb, obb = ib[b], ob[b]

      @plsc.parallel_loop(0, rpc // 5, 1, unroll=2)
      def rowblk(r0, ibb=ibb, obb=obb):
        rs = [r0 * 5 + i for i in range(5)]
        xs = [ibb[r, pl.ds(0, hw)] for r in rs]
        ys = [ibb[r, pl.ds(hw, hw)] for r in rs]
        ws = [plsc.bitcast(
            plsc.pack(x, y, format=plsc.PackFormat.INTERLEAVED), jnp.int32)
            for x, y in zip(xs, ys)]
        for r, w in zip(rs, ws):
          obb[r, pl.ds(0, hw)] = w
      pltpu.make_async_copy(
          ob[b], dsts[t].at[pl.ds(base + c * rpc, rpc)], osem[b]).start()

    for b in (0, 1):
      pltpu.make_async_copy(
          ob[b], dsts[0].at[pl.ds(0, rpc)], osem[b]).wait()

  return pk


def _make_main_kernel(n, d, e):
  hw = d // 2
  nch = e // _C
  base_n = nch // _NW
  extra = nch - base_n * _NW
  npair = (base_n + 2) // 2

  @functools.partial(
      pl.kernel, mesh=_MESH, compiler_params=_PARAMS,
      out_type=jax.ShapeDtypeStruct((e,), jnp.float32),
      scratch_types=[
          pltpu.VMEM((_C,), jnp.int32),
          pltpu.VMEM((_C,), jnp.int32),
          pltpu.VMEM((_C,), jnp.int32),
          pltpu.VMEM((_C,), jnp.int32),
          pltpu.VMEM((_C, hw), jnp.int32),
          pltpu.VMEM((_C, hw), jnp.int32),
          pltpu.VMEM((_C, hw), jnp.int32),
          pltpu.VMEM((_C, hw), jnp.int32),
          pltpu.VMEM((_C,), jnp.float32),
          pltpu.VMEM((_C,), jnp.float32),
          pltpu.VMEM((4, 16, 17), jnp.float32),
          pltpu.SemaphoreType.DMA,
          pltpu.SemaphoreType.DMA,
          pltpu.SemaphoreType.DMA,
          pltpu.SemaphoreType.DMA,
          pltpu.SemaphoreType.DMA,
          pltpu.SemaphoreType.DMA,
      ],
  )
  def k(ei_hbm, pu_hbm, pv_hbm, out_hbm,
        iu0, iu1, iv0, iv1, ru0, ru1, rv0, rv1, ov0, ov1, s1,
        is0, is1, rs0, rs1, os0, os1):
    iu = [iu0, iu1]
    iv = [iv0, iv1]
    ru = [ru0, ru1]
    rv = [rv0, rv1]
    ov = [ov0, ov1]
    isem = [is0, is1]
    rsem = [rs0, rs1]
    osem = [os0, os1]

    wid = _worker_id()
    n_i = jnp.where(wid < extra, base_n + 1, base_n)
    lanes = lax.iota(jnp.int32, 16)

    def edge0(ci):
      return (wid + ci * _NW) * _C

    def fetch_idx(ci, b):
      eb = edge0(ci)
      pltpu.make_async_copy(
          ei_hbm.at[0, pl.ds(eb, _C)], iu[b], isem[b]).start()
      pltpu.make_async_copy(
          ei_hbm.at[1, pl.ds(eb, _C)], iv[b], isem[b]).start()

    def wait_idx(b):
      pltpu.make_async_copy(ei_hbm.at[0, pl.ds(0, _C)], iu[b], isem[b]).wait()
      pltpu.make_async_copy(ei_hbm.at[1, pl.ds(0, _C)], iv[b], isem[b]).wait()

    def fetch_rows(b):
      for j in range(_R):
        sl = pl.ds(j * 128, 128)
        pltpu.make_async_copy(
            pu_hbm.at[iu[b].at[sl]], ru[b].at[sl], rsem[b]).start()
        pltpu.make_async_copy(
            pv_hbm.at[iv[b].at[sl]], rv[b].at[sl], rsem[b]).start()

    def wait_rows(b):
      pltpu.make_async_copy(pu_hbm.at[pl.ds(0, _C)], ru[b], rsem[b]).wait()
      pltpu.make_async_copy(pv_hbm.at[pl.ds(0, _C)], rv[b], rsem[b]).wait()

    def compute_store(ci, b):
      rub, rvb, ovb = ru[b], rv[b], ov[b]

      @plsc.parallel_loop(0, _C // 16, 1, unroll=4)
      def group(g):
        e0 = g * 16
        t = g & 3
        s1t = s1.at[t]
        for blk in range(4):
          es = [e0 + blk * 4 + i for i in range(4)]
          us = [rub[e, pl.ds(0, hw)] for e in es]
          vs = [rvb[e, pl.ds(0, hw)] for e in es]
          ps = [plsc.bitcast(u, jnp.bfloat16) * plsc.bitcast(v, jnp.bfloat16)
                for u, v in zip(us, vs)]
          hs = [plsc.unpack(p, format=plsc.PackFormat.INTERLEAVED)
                for p in ps]
          for i in range(4):
            s1t[blk * 4 + i, pl.ds(0, 16)] = hs[i][0] + hs[i][1]
        gs = [plsc.load_gather(s1t, [lanes, jnp.full((16,), dd, jnp.int32)])
              for dd in range(16)]
        while len(gs) > 1:
          gs = [gs[i] + gs[i + 1] for i in range(0, len(gs), 2)]
        ovb[pl.ds(e0, 16)] = 1.0 / (1.0 + jnp.exp(-gs[0]))
      pltpu.make_async_copy(
          ov[b], out_hbm.at[pl.ds(edge0(ci), _C)], osem[b]).start()

    def wait_out(b):
      pltpu.make_async_copy(
          ov[b], out_hbm.at[pl.ds(0, _C)], osem[b]).wait()

    fetch_idx(0, 0)
    wait_idx(0)
    fetch_rows(0)

    @pl.when(1 < n_i)
    def _():
      fetch_idx(1, 1)

    def pair(p, carry):
      for b in (0, 1):
        ci = p * 2 + b
        nb = 1 - b

        @pl.when(ci < n_i)
        def _():
          @pl.when(ci + 1 < n_i)
          def _():
            wait_idx(nb)
            fetch_rows(nb)

          wait_rows(b)

          @pl.when(ci + 2 < n_i)
          def _():
            fetch_idx(ci + 2, b)

          @pl.when(ci >= 2)
          def _():
            wait_out(b)

          compute_store(ci, b)
      return carry

    lax.fori_loop(0, npair, pair, 0)
    wait_out(0)
    wait_out(1)

  return k


def kernel(Eu, Ev, edge_index):
  n, d = Eu.shape
  e = edge_index.shape[1]
  pu, pv = _make_pack_kernel(n, d)(Eu, Ev)
  return _make_main_kernel(n, d, e)(edge_index.astype(jnp.int32), pu, pv)
